# Initial kernel scaffold; baseline (speedup 1.0000x reference)
import jax
import jax.numpy as jnp
from jax.experimental import pallas as pl


def kernel(x, W, b, mask_token):
    raise NotImplementedError("write your pallas kernel here")


# speedup vs baseline: 2.6548x; 2.6548x over previous
import functools
import math

import jax
import jax.numpy as jnp
import numpy as np
from jax.experimental import pallas as pl
from jax.experimental.pallas import tpu as pltpu

_PATCH = 16
_EMBED = 768
_HW = 512
_HP = _HW // _PATCH
_NPATCH = _HP * _HP
_WIN = 7
_NWIN = 4
_RATIO = 0.8
_EBLK = 256


def _masked_rows():
    H = W_ = _HP
    all_inds = np.arange(H * W_, dtype=np.int32).reshape(H, W_)
    pad = _WIN // 2
    selectable = all_inds[pad:-pad, pad:-pad].reshape(-1)
    key = jax.random.key(42)
    sampled_idx = np.asarray(
        jax.random.choice(key, selectable.shape[0], (_NWIN,), replace=False))
    centroids = selectable[sampled_idx]
    off = np.arange(int(math.ceil(-_WIN / 2)), int(math.ceil(_WIN / 2)),
                    dtype=np.int32)
    window_offsets = np.tile(off[None, :], (_WIN, 1))
    squaring = np.tile((off * H)[None, :], (_WIN, 1)).T
    window_offsets = (window_offsets + squaring).reshape(1, -1)
    coords = np.tile(centroids[:, None], (1, _WIN ** 2)) + window_offsets
    n_mask = int(_RATIO * _WIN ** 2)
    return tuple(tuple(int(r) for r in row) for row in coords[:, :n_mask])


_ROWS = _masked_rows()


def _mae_kernel(rows, xp_ref, wt_ref, b_ref, mt_ref, out_ref):
    y = jnp.dot(xp_ref[0], wt_ref[...], preferred_element_type=jnp.float32)
    y = y + b_ref[...]
    mt = mt_ref[0]
    for w in range(_NWIN):
        out_ref[0, w] = y
    for w in range(_NWIN):
        for r in rows[w]:
            out_ref[0, w, r] = mt


def kernel(x, W, b, mask_token):
    Bn = x.shape[0]
    xp = x.reshape(Bn, 3, _HP, _PATCH, _HP, _PATCH)
    xp = xp.transpose(0, 2, 4, 1, 3, 5).reshape(Bn, _NPATCH, 3 * _PATCH * _PATCH)
    wt = W.reshape(_EMBED, 3 * _PATCH * _PATCH).T
    b2 = b.reshape(1, _EMBED)
    mt2 = mask_token.reshape(1, _EMBED)

    rows = _ROWS
    n_eblk = _EMBED // _EBLK
    out = pl.pallas_call(
        functools.partial(_mae_kernel, rows),
        grid=(Bn, n_eblk),
        in_specs=[
            pl.BlockSpec((1, _NPATCH, 3 * _PATCH * _PATCH),
                         lambda i, e: (i, 0, 0)),
            pl.BlockSpec((3 * _PATCH * _PATCH, _EBLK), lambda i, e: (0, e)),
            pl.BlockSpec((1, _EBLK), lambda i, e: (0, e)),
            pl.BlockSpec((1, _EBLK), lambda i, e: (0, e)),
        ],
        out_specs=pl.BlockSpec((1, _NWIN, _NPATCH, _EBLK),
                               lambda i, e: (i, 0, 0, e)),
        out_shape=jax.ShapeDtypeStruct((Bn, _NWIN, _NPATCH, _EMBED),
                                       jnp.float32),
        compiler_params=pltpu.CompilerParams(
            dimension_semantics=("parallel", "parallel")),
    )(xp, wt, b2, mt2)
    return out


# speedup vs baseline: 2.6571x; 1.0008x over previous
import functools
import math

import jax
import jax.numpy as jnp
import numpy as np
from jax.experimental import pallas as pl
from jax.experimental.pallas import tpu as pltpu

_PATCH = 16
_EMBED = 768
_HW = 512
_HP = _HW // _PATCH
_NPATCH = _HP * _HP
_WIN = 7
_NWIN = 4
_RATIO = 0.8
_EBLK = 256


_ROWS = (
    (145, 146, 147, 148, 149, 150, 151, 177, 178, 179, 180, 181, 182, 183,
     209, 210, 211, 212, 213, 214, 215, 241, 242, 243, 244, 245, 246, 247,
     273, 274, 275, 276, 277, 278, 279, 305, 306, 307, 308),
    (755, 756, 757, 758, 759, 760, 761, 787, 788, 789, 790, 791, 792, 793,
     819, 820, 821, 822, 823, 824, 825, 851, 852, 853, 854, 855, 856, 857,
     883, 884, 885, 886, 887, 888, 889, 915, 916, 917, 918),
    (588, 589, 590, 591, 592, 593, 594, 620, 621, 622, 623, 624, 625, 626,
     652, 653, 654, 655, 656, 657, 658, 684, 685, 686, 687, 688, 689, 690,
     716, 717, 718, 719, 720, 721, 722, 748, 749, 750, 751),
    (41, 42, 43, 44, 45, 46, 47, 73, 74, 75, 76, 77, 78, 79,
     105, 106, 107, 108, 109, 110, 111, 137, 138, 139, 140, 141, 142, 143,
     169, 170, 171, 172, 173, 174, 175, 201, 202, 203, 204),
)


def _runs(rows):
    out = []
    for r in rows:
        if out and out[-1][0] + out[-1][1] == r:
            out[-1] = (out[-1][0], out[-1][1] + 1)
        else:
            out.append((r, 1))
    return tuple(out)


_ROW_RUNS = tuple(_runs(sorted(rows)) for rows in _ROWS)


def _mae_kernel(runs, xp_ref, wt_ref, b_ref, mt_ref, out_ref):
    y = jnp.dot(xp_ref[0], wt_ref[...], preferred_element_type=jnp.float32)
    y = y + b_ref[...]
    for w in range(_NWIN):
        out_ref[0, w] = y
    for w in range(_NWIN):
        for start, length in runs[w]:
            out_ref[0, w, pl.ds(start, length)] = jnp.broadcast_to(
                mt_ref[...], (length, mt_ref.shape[1]))


def kernel(x, W, b, mask_token):
    Bn = x.shape[0]
    xp = x.reshape(Bn, 3, _HP, _PATCH, _HP, _PATCH)
    xp = xp.transpose(0, 2, 4, 1, 3, 5).reshape(Bn, _NPATCH, 3 * _PATCH * _PATCH)
    wt = W.reshape(_EMBED, 3 * _PATCH * _PATCH).T
    b2 = b.reshape(1, _EMBED)
    mt2 = mask_token.reshape(1, _EMBED)

    n_eblk = _EMBED // _EBLK
    out = pl.pallas_call(
        functools.partial(_mae_kernel, _ROW_RUNS),
        grid=(Bn, n_eblk),
        in_specs=[
            pl.BlockSpec((1, _NPATCH, 3 * _PATCH * _PATCH),
                         lambda i, e: (i, 0, 0)),
            pl.BlockSpec((3 * _PATCH * _PATCH, _EBLK), lambda i, e: (0, e)),
            pl.BlockSpec((1, _EBLK), lambda i, e: (0, e)),
            pl.BlockSpec((1, _EBLK), lambda i, e: (0, e)),
        ],
        out_specs=pl.BlockSpec((1, _NWIN, _NPATCH, _EBLK),
                               lambda i, e: (i, 0, 0, e)),
        out_shape=jax.ShapeDtypeStruct((Bn, _NWIN, _NPATCH, _EMBED),
                                       jnp.float32),
        compiler_params=pltpu.CompilerParams(
            dimension_semantics=("parallel", "parallel")),
    )(xp, wt, b2, mt2)
    return out


# speedup vs baseline: 2.9859x; 1.1238x over previous
import functools
import math

import jax
import jax.numpy as jnp
import numpy as np
from jax.experimental import pallas as pl
from jax.experimental.pallas import tpu as pltpu

_PATCH = 16
_EMBED = 768
_HW = 512
_HP = _HW // _PATCH
_NPATCH = _HP * _HP
_WIN = 7
_NWIN = 4
_RATIO = 0.8
_EBLK = 256


_ROWS = (
    (145, 146, 147, 148, 149, 150, 151, 177, 178, 179, 180, 181, 182, 183,
     209, 210, 211, 212, 213, 214, 215, 241, 242, 243, 244, 245, 246, 247,
     273, 274, 275, 276, 277, 278, 279, 305, 306, 307, 308),
    (755, 756, 757, 758, 759, 760, 761, 787, 788, 789, 790, 791, 792, 793,
     819, 820, 821, 822, 823, 824, 825, 851, 852, 853, 854, 855, 856, 857,
     883, 884, 885, 886, 887, 888, 889, 915, 916, 917, 918),
    (588, 589, 590, 591, 592, 593, 594, 620, 621, 622, 623, 624, 625, 626,
     652, 653, 654, 655, 656, 657, 658, 684, 685, 686, 687, 688, 689, 690,
     716, 717, 718, 719, 720, 721, 722, 748, 749, 750, 751),
    (41, 42, 43, 44, 45, 46, 47, 73, 74, 75, 76, 77, 78, 79,
     105, 106, 107, 108, 109, 110, 111, 137, 138, 139, 140, 141, 142, 143,
     169, 170, 171, 172, 173, 174, 175, 201, 202, 203, 204),
)


def _runs(rows):
    out = []
    for r in rows:
        if out and out[-1][0] + out[-1][1] == r:
            out[-1] = (out[-1][0], out[-1][1] + 1)
        else:
            out.append((r, 1))
    return tuple(out)


_ROW_RUNS = tuple(_runs(sorted(rows)) for rows in _ROWS)


def _mae_kernel(runs, xp_ref, wt_ref, b_ref, mt_ref, out_ref):
    y = jnp.dot(xp_ref[0], wt_ref[...], preferred_element_type=jnp.float32)
    y = y + b_ref[...]
    for w in range(_NWIN):
        out_ref[0, w] = y
    for w in range(_NWIN):
        for start, length in runs[w]:
            out_ref[0, w, pl.ds(start, length)] = jnp.broadcast_to(
                mt_ref[...], (length, mt_ref.shape[1]))


def kernel(x, W, b, mask_token):
    Bn = x.shape[0]
    xp = x.reshape(Bn, 3, _HP, _PATCH, _HP, _PATCH)
    xp = xp.transpose(0, 2, 4, 1, 3, 5).reshape(Bn, _NPATCH, 3 * _PATCH * _PATCH)
    xp = xp.astype(jnp.bfloat16)
    wt = W.reshape(_EMBED, 3 * _PATCH * _PATCH).T.astype(jnp.bfloat16)
    b2 = b.reshape(1, _EMBED)
    mt2 = mask_token.reshape(1, _EMBED)

    n_eblk = _EMBED // _EBLK
    out = pl.pallas_call(
        functools.partial(_mae_kernel, _ROW_RUNS),
        grid=(Bn, n_eblk),
        in_specs=[
            pl.BlockSpec((1, _NPATCH, 3 * _PATCH * _PATCH),
                         lambda i, e: (i, 0, 0)),
            pl.BlockSpec((3 * _PATCH * _PATCH, _EBLK), lambda i, e: (0, e)),
            pl.BlockSpec((1, _EBLK), lambda i, e: (0, e)),
            pl.BlockSpec((1, _EBLK), lambda i, e: (0, e)),
        ],
        out_specs=pl.BlockSpec((1, _NWIN, _NPATCH, _EBLK),
                               lambda i, e: (i, 0, 0, e)),
        out_shape=jax.ShapeDtypeStruct((Bn, _NWIN, _NPATCH, _EMBED),
                                       jnp.float32),
        compiler_params=pltpu.CompilerParams(
            dimension_semantics=("parallel", "parallel")),
    )(xp, wt, b2, mt2)
    return out


# speedup vs baseline: 3.1159x; 1.0435x over previous
import functools
import math

import jax
import jax.numpy as jnp
import numpy as np
from jax.experimental import pallas as pl
from jax.experimental.pallas import tpu as pltpu

_PATCH = 16
_EMBED = 768
_HW = 512
_HP = _HW // _PATCH
_NPATCH = _HP * _HP
_WIN = 7
_NWIN = 4
_RATIO = 0.8
_EBLK = 768


_ROWS = (
    (145, 146, 147, 148, 149, 150, 151, 177, 178, 179, 180, 181, 182, 183,
     209, 210, 211, 212, 213, 214, 215, 241, 242, 243, 244, 245, 246, 247,
     273, 274, 275, 276, 277, 278, 279, 305, 306, 307, 308),
    (755, 756, 757, 758, 759, 760, 761, 787, 788, 789, 790, 791, 792, 793,
     819, 820, 821, 822, 823, 824, 825, 851, 852, 853, 854, 855, 856, 857,
     883, 884, 885, 886, 887, 888, 889, 915, 916, 917, 918),
    (588, 589, 590, 591, 592, 593, 594, 620, 621, 622, 623, 624, 625, 626,
     652, 653, 654, 655, 656, 657, 658, 684, 685, 686, 687, 688, 689, 690,
     716, 717, 718, 719, 720, 721, 722, 748, 749, 750, 751),
    (41, 42, 43, 44, 45, 46, 47, 73, 74, 75, 76, 77, 78, 79,
     105, 106, 107, 108, 109, 110, 111, 137, 138, 139, 140, 141, 142, 143,
     169, 170, 171, 172, 173, 174, 175, 201, 202, 203, 204),
)


def _runs(rows):
    out = []
    for r in rows:
        if out and out[-1][0] + out[-1][1] == r:
            out[-1] = (out[-1][0], out[-1][1] + 1)
        else:
            out.append((r, 1))
    return tuple(out)


_ROW_RUNS = tuple(_runs(sorted(rows)) for rows in _ROWS)


def _mae_kernel(runs, xp_ref, wt_ref, b_ref, mt_ref, out_ref):
    y = jnp.dot(xp_ref[0], wt_ref[...], preferred_element_type=jnp.float32)
    y = y + b_ref[...]
    for w in range(_NWIN):
        out_ref[0, w] = y
    for w in range(_NWIN):
        for start, length in runs[w]:
            out_ref[0, w, pl.ds(start, length)] = jnp.broadcast_to(
                mt_ref[...], (length, mt_ref.shape[1]))


def kernel(x, W, b, mask_token):
    Bn = x.shape[0]
    xp = x.reshape(Bn, 3, _HP, _PATCH, _HP, _PATCH)
    xp = xp.transpose(0, 2, 4, 1, 3, 5).reshape(Bn, _NPATCH, 3 * _PATCH * _PATCH)
    xp = xp.astype(jnp.bfloat16)
    wt = W.reshape(_EMBED, 3 * _PATCH * _PATCH).T.astype(jnp.bfloat16)
    b2 = b.reshape(1, _EMBED)
    mt2 = mask_token.reshape(1, _EMBED)

    n_eblk = _EMBED // _EBLK
    out = pl.pallas_call(
        functools.partial(_mae_kernel, _ROW_RUNS),
        grid=(Bn, n_eblk),
        in_specs=[
            pl.BlockSpec((1, _NPATCH, 3 * _PATCH * _PATCH),
                         lambda i, e: (i, 0, 0)),
            pl.BlockSpec((3 * _PATCH * _PATCH, _EBLK), lambda i, e: (0, e)),
            pl.BlockSpec((1, _EBLK), lambda i, e: (0, e)),
            pl.BlockSpec((1, _EBLK), lambda i, e: (0, e)),
        ],
        out_specs=pl.BlockSpec((1, _NWIN, _NPATCH, _EBLK),
                               lambda i, e: (i, 0, 0, e)),
        out_shape=jax.ShapeDtypeStruct((Bn, _NWIN, _NPATCH, _EMBED),
                                       jnp.float32),
        compiler_params=pltpu.CompilerParams(
            dimension_semantics=("parallel", "parallel")),
    )(xp, wt, b2, mt2)
    return out
